# Initial kernel scaffold; baseline (speedup 1.0000x reference)
import jax
import jax.numpy as jnp
from jax.experimental import pallas as pl


def kernel(x, scores, expert_weights, top_experts, W1, W2):
    raise NotImplementedError("write your pallas kernel here")


# speedup vs baseline: 2.9556x; 2.9556x over previous
import functools

import jax
import jax.numpy as jnp
from jax import lax
from jax.experimental import pallas as pl
from jax.experimental.pallas import tpu as pltpu
from jax.experimental.pallas import tpu_sc as plsc

H = 1024
FFN = 2048
E = 16
T = 4096
K = 2
P = T * K
BLK = 256
NB = P // BLK + E
PADP = NB * BLK

NC, NS = 2, 16
NW = NC * NS
CH = 64


@functools.lru_cache(maxsize=None)
def _make_row_gather(n_rows, d):
    assert n_rows % (NW * CH) == 0
    per_w = n_rows // NW
    n_ch = per_w // CH
    mesh = plsc.VectorSubcoreMesh(core_axis_name="c", subcore_axis_name="s")

    @functools.partial(
        pl.kernel,
        mesh=mesh,
        out_type=jax.ShapeDtypeStruct((n_rows, d), jnp.float32),
        scratch_types=[
            pltpu.VMEM((CH,), jnp.int32),
            pltpu.VMEM((CH, d), jnp.float32),
            pltpu.SemaphoreType.DMA,
        ],
    )
    def gather_rows(src_hbm, idx_hbm, out_hbm, idx_v, rows_v, sem):
        wid = lax.axis_index("s") * NC + lax.axis_index("c")
        base = wid * per_w
        for c in range(n_ch):
            off = base + c * CH
            pltpu.sync_copy(idx_hbm.at[pl.ds(off, CH)], idx_v)
            pltpu.async_copy(src_hbm.at[idx_v], rows_v, sem).wait()
            pltpu.sync_copy(rows_v, out_hbm.at[pl.ds(off, CH)])

    return gather_rows


def _gelu(h):
    return 0.5 * h * (1.0 + lax.erf(h * 0.7071067811865476))


def _gmm_body(be_ref, x_ref, w1_ref, w2_ref, o_ref):
    h = lax.dot_general(x_ref[...], w1_ref[0], (((1,), (1,)), ((), ())),
                        preferred_element_type=jnp.float32)
    h = _gelu(h)
    o_ref[...] = lax.dot_general(h, w2_ref[0], (((1,), (0,)), ((), ())),
                                 preferred_element_type=jnp.float32)


def _gmm(block_expert, xg, w1, w2):
    grid_spec = pltpu.PrefetchScalarGridSpec(
        num_scalar_prefetch=1,
        grid=(NB,),
        in_specs=[
            pl.BlockSpec((BLK, H), lambda b, be: (b, 0)),
            pl.BlockSpec((1, FFN, H), lambda b, be: (be[b], 0, 0)),
            pl.BlockSpec((1, FFN, H), lambda b, be: (be[b], 0, 0)),
        ],
        out_specs=pl.BlockSpec((BLK, H), lambda b, be: (b, 0)),
    )
    return pl.pallas_call(
        _gmm_body,
        grid_spec=grid_spec,
        out_shape=jax.ShapeDtypeStruct((PADP, H), jnp.float32),
    )(block_expert, xg, w1, w2)


def _combine_body(ga_ref, gb_ref, w0_ref, w1_ref, o_ref):
    w0 = w0_ref[0, 0, :]
    w1 = w1_ref[0, 0, :]
    o_ref[...] = ga_ref[...] * w0[:, None] + gb_ref[...] * w1[:, None]


def _combine(garr, g0, g1):
    nb = T // BLK
    return pl.pallas_call(
        _combine_body,
        grid=(nb,),
        in_specs=[
            pl.BlockSpec((BLK, H), lambda b: (b, 0)),
            pl.BlockSpec((BLK, H), lambda b: (b + nb, 0)),
            pl.BlockSpec((1, 1, BLK), lambda b: (b, 0, 0)),
            pl.BlockSpec((1, 1, BLK), lambda b: (b, 0, 0)),
        ],
        out_specs=pl.BlockSpec((BLK, H), lambda b: (b, 0)),
        out_shape=jax.ShapeDtypeStruct((T, H), jnp.float32),
    )(garr, garr, g0.reshape(nb, 1, BLK), g1.reshape(nb, 1, BLK))


def _routing(top_experts):
    e_flat = top_experts.reshape(P)
    order = jnp.argsort(e_flat).astype(jnp.int32)
    sorted_e = e_flat[order]
    counts = jnp.bincount(e_flat, length=E)
    blocks_pe = (counts + BLK - 1) // BLK
    blk_start = jnp.concatenate(
        [jnp.zeros(1, blocks_pe.dtype), jnp.cumsum(blocks_pe)[:-1]])
    grp_start = jnp.concatenate(
        [jnp.zeros(1, counts.dtype), jnp.cumsum(counts)[:-1]])
    blk_end = blk_start + blocks_pe
    b_ids = jnp.arange(NB)
    block_expert = jnp.minimum(
        jnp.sum(b_ids[:, None] >= blk_end[None, :], axis=1), E - 1
    ).astype(jnp.int32)
    r_ids = jnp.arange(PADP)
    r_e = block_expert[r_ids // BLK]
    off = r_ids - blk_start[r_e] * BLK
    valid = off < counts[r_e]
    pair = order[jnp.where(valid, grp_start[r_e] + off, 0)]
    src_tok = jnp.where(valid, pair // K, 0).astype(jnp.int32)
    pos_sorted = (blk_start[sorted_e] * BLK
                  + (jnp.arange(P) - grp_start[sorted_e])).astype(jnp.int32)
    rpos = jnp.zeros(P, jnp.int32).at[order].set(pos_sorted).reshape(T, K)
    ridx = jnp.concatenate([rpos[:, 0], rpos[:, 1]])
    return block_expert, src_tok, ridx


def kernel(x, scores, expert_weights, top_experts, W1, W2):
    del scores
    w1 = W1.reshape(E, FFN, H)
    w2 = W2.reshape(E, FFN, H)
    block_expert, src_tok, ridx = _routing(top_experts)
    xg = _make_row_gather(PADP, H)(x, src_tok)
    og = _gmm(block_expert, xg, w1, w2)
    garr = _make_row_gather(P, H)(og, ridx)
    return _combine(garr, expert_weights[:, 0], expert_weights[:, 1])
